# Initial kernel scaffold; baseline (speedup 1.0000x reference)
import jax
import jax.numpy as jnp
from jax.experimental import pallas as pl


def kernel(x, batch, W1, b1, W2, b2):
    raise NotImplementedError("write your pallas kernel here")


# speedup vs baseline: 17.4792x; 17.4792x over previous
import jax
import jax.numpy as jnp
from jax.experimental import pallas as pl
from jax.experimental.pallas import tpu as pltpu

N = 50000
D = 256
HID = 128
NSEG = 512
R = 2048
GRID = (N + R - 1) // R


def _body(x_ref, b_ref, W1_ref, b1_ref, W2_ref, b2_ref, out_ref, acc_ref, ssum_ref):
    i = pl.program_id(0)
    xb = x_ref[...]
    bb = b_ref[...]
    W1 = W1_ref[...]
    W2 = W2_ref[...]
    h = jnp.tanh(jnp.dot(xb, W1, preferred_element_type=jnp.float32)
                 + b1_ref[...][None, :])
    s = jnp.dot(h, W2, preferred_element_type=jnp.float32) + b2_ref[...][None, :]
    shift = jnp.sum(jnp.abs(W2)) + jnp.abs(b2_ref[0])
    rowid = i * R + jax.lax.broadcasted_iota(jnp.int32, (R, 1), 0)
    valid = rowid < N
    u = jnp.where(valid, jnp.exp(s - shift), 0.0)
    wx = jnp.where(valid, xb * u, 0.0)
    seg = jax.lax.broadcasted_iota(jnp.int32, (NSEG, R), 0)
    P = (seg == bb[None, :]).astype(jnp.float32)
    acc_part = jnp.dot(P, wx, preferred_element_type=jnp.float32)
    s_part = jnp.dot(P, u, preferred_element_type=jnp.float32)

    @pl.when(i == 0)
    def _init():
        acc_ref[...] = jnp.zeros_like(acc_ref)
        ssum_ref[...] = jnp.zeros_like(ssum_ref)

    acc_ref[...] += acc_part
    ssum_ref[...] += s_part

    @pl.when(i == GRID - 1)
    def _fin():
        out_ref[...] = acc_ref[...] / (ssum_ref[...] + 1e-30)


def kernel(x, batch, W1, b1, W2, b2):
    batch = batch.astype(jnp.int32)
    return pl.pallas_call(
        _body,
        grid=(GRID,),
        in_specs=[
            pl.BlockSpec((R, D), lambda i: (i, 0)),
            pl.BlockSpec((R,), lambda i: (i,)),
            pl.BlockSpec((D, HID), lambda i: (0, 0)),
            pl.BlockSpec((HID,), lambda i: (0,)),
            pl.BlockSpec((HID, 1), lambda i: (0, 0)),
            pl.BlockSpec((1,), lambda i: (0,)),
        ],
        out_specs=pl.BlockSpec((NSEG, D), lambda i: (0, 0)),
        out_shape=jax.ShapeDtypeStruct((NSEG, D), jnp.float32),
        scratch_shapes=[
            pltpu.VMEM((NSEG, D), jnp.float32),
            pltpu.VMEM((NSEG, 1), jnp.float32),
        ],
    )(x, batch, W1, b1, W2, b2)


# speedup vs baseline: 17.5275x; 1.0028x over previous
import jax
import jax.numpy as jnp
from jax.experimental import pallas as pl
from jax.experimental.pallas import tpu as pltpu

N = 50000
D = 256
HID = 128
NSEG = 512
R = 2048
GRID = (N + R - 1) // R


def _body(x_ref, b_ref, W1_ref, b1_ref, W2_ref, b2_ref, out_ref, acc_ref, ssum_ref):
    i = pl.program_id(0)
    xb = x_ref[...]
    bb = b_ref[...]
    W1 = W1_ref[...]
    W2 = W2_ref[...]
    h = jnp.tanh(jnp.dot(xb, W1, preferred_element_type=jnp.float32)
                 + b1_ref[...][None, :])
    s = jnp.dot(h, W2, preferred_element_type=jnp.float32) + b2_ref[...][None, :]
    shift = jnp.sum(jnp.abs(W2)) + jnp.abs(b2_ref[0])
    rowid = i * R + jax.lax.broadcasted_iota(jnp.int32, (R, 1), 0)
    valid = rowid < N
    u = jnp.where(valid, jnp.exp(s - shift), 0.0)
    wx = jnp.where(valid, xb * u, 0.0)
    seg = jax.lax.broadcasted_iota(jnp.int32, (NSEG, R), 0)
    P = (seg == bb[None, :]).astype(jnp.bfloat16)
    acc_part = jnp.dot(P, wx.astype(jnp.bfloat16),
                       preferred_element_type=jnp.float32)
    s_part = jnp.dot(P, u.astype(jnp.bfloat16),
                     preferred_element_type=jnp.float32)

    @pl.when(i == 0)
    def _init():
        acc_ref[...] = jnp.zeros_like(acc_ref)
        ssum_ref[...] = jnp.zeros_like(ssum_ref)

    acc_ref[...] += acc_part
    ssum_ref[...] += s_part

    @pl.when(i == GRID - 1)
    def _fin():
        out_ref[...] = acc_ref[...] / (ssum_ref[...] + 1e-30)


def kernel(x, batch, W1, b1, W2, b2):
    batch = batch.astype(jnp.int32)
    return pl.pallas_call(
        _body,
        grid=(GRID,),
        in_specs=[
            pl.BlockSpec((R, D), lambda i: (i, 0)),
            pl.BlockSpec((R,), lambda i: (i,)),
            pl.BlockSpec((D, HID), lambda i: (0, 0)),
            pl.BlockSpec((HID,), lambda i: (0,)),
            pl.BlockSpec((HID, 1), lambda i: (0, 0)),
            pl.BlockSpec((1,), lambda i: (0,)),
        ],
        out_specs=pl.BlockSpec((NSEG, D), lambda i: (0, 0)),
        out_shape=jax.ShapeDtypeStruct((NSEG, D), jnp.float32),
        scratch_shapes=[
            pltpu.VMEM((NSEG, D), jnp.float32),
            pltpu.VMEM((NSEG, 1), jnp.float32),
        ],
    )(x, batch, W1, b1, W2, b2)
